# baseline (speedup 1.0000x reference)
import functools

import jax
import jax.numpy as jnp
from jax import lax
from jax.experimental import pallas as pl
from jax.experimental.pallas import tpu as pltpu
from jax.experimental.pallas import tpu_sc as plsc

_K = 0


def kernel(u):
    B, S, D = u.shape

    mesh = plsc.ScalarSubcoreMesh(axis_name="c", num_cores=1)

    @functools.partial(
        pl.kernel,
        mesh=mesh,
        out_type=jax.ShapeDtypeStruct((B, 1, D), u.dtype),
        scratch_types=[pltpu.SemaphoreType.DMA],
    )
    def sc_row_gather(u_hbm, o_hbm, sem):
        copy = pltpu.make_async_copy(u_hbm.at[:, _K], o_hbm.at[:, 0], sem)
        copy.start()
        copy.wait()

    return sc_row_gather(u)
